# baseline (speedup 1.0000x reference)
import functools

import jax
import jax.numpy as jnp
from jax.experimental import pallas as pl
from jax.experimental.pallas import tpu as pltpu

_TR = 256


def _qmix_block(bb, tr, q_ref, s_ref, wcat_ref, qw_ref, b2w_ref, b2b_ref,
                out_ref):
    f32 = jnp.float32
    bf16 = jnp.bfloat16

    wc = wcat_ref[...]
    w = jnp.concatenate([
        wc[:, 0:512],
        wc[:, 512:576],
        0.5 * wc[:, 640:704],
        wc[:, 768:896],
    ], axis=1).astype(bf16)
    qw = qw_ref[...].astype(bf16)
    ones_red = jnp.ones((128, 8), bf16)
    b2w = b2w_ref[...]
    b2b = b2b_ref[...]

    s1 = jnp.concatenate(
        [s_ref[...].astype(bf16), jnp.ones((bb, 1), bf16)], axis=1)
    q1 = q_ref[...].astype(bf16)
    low = jax.lax.broadcasted_iota(jnp.int32, (tr, 128), 1) < 64

    for t in range(bb // tr):
        r0, r1 = t * tr, (t + 1) * tr
        p = jnp.dot(s1[r0:r1, :], w, preferred_element_type=f32)
        qe = jnp.dot(q1[r0:r1, :], qw, preferred_element_type=f32)

        y = None
        for j in range(4):
            x = jnp.abs(p[:, 128 * j:128 * (j + 1)])
            x = x * qe[:, 128 * j:128 * (j + 1)]
            y = x if y is None else y + x
        hid = y + pltpu.roll(y, 64, axis=1)

        bw = p[:, 512:640]
        r = pltpu.roll(bw, 64, axis=1)
        b1d = jnp.where(low, bw, r)
        w2d = jnp.abs(jnp.where(low, r, bw))

        mixed = hid + b1d
        mixed = jnp.where(mixed > 0.0, mixed,
                          jnp.exp(jnp.minimum(mixed, 0.0)) - 1.0)

        h2 = jnp.maximum(p[:, 640:768], 0.0)
        z = (mixed * w2d + h2 * b2w).astype(bf16)
        res = jnp.dot(z, ones_red, preferred_element_type=f32)
        out_ref[r0:r1, :] = res[:, 0:1] + b2b


def kernel(qagents, state, w_cat, expand, reduce, b2w, b2b):
    del reduce
    f32 = jnp.float32
    B, A = qagents.shape
    S = state.shape[1]
    Sk, c = w_cat.shape
    w0 = expand.shape[1]

    BB = 4096 if B % 4096 == 0 else max(8, ((B + 7) // 8) * 8)
    TR = _TR if BB % _TR == 0 else BB
    grid_b = pl.cdiv(B, BB)
    b_pad = grid_b * BB
    if b_pad != B:
        qagents = jnp.pad(qagents, ((0, b_pad - B), (0, 0)))
        state = jnp.pad(state, ((0, b_pad - B), (0, 0)))

    out = pl.pallas_call(
        functools.partial(_qmix_block, BB, TR),
        out_shape=jax.ShapeDtypeStruct((b_pad, 1), f32),
        grid=(grid_b,),
        in_specs=[
            pl.BlockSpec((BB, A), lambda i: (i, 0)),
            pl.BlockSpec((BB, S), lambda i: (i, 0)),
            pl.BlockSpec((Sk, c), lambda i: (0, 0)),
            pl.BlockSpec((A, w0), lambda i: (0, 0)),
            pl.BlockSpec((1, 128), lambda i: (0, 0)),
            pl.BlockSpec((1, 1), lambda i: (0, 0)),
        ],
        out_specs=pl.BlockSpec((BB, 1), lambda i: (i, 0)),
        compiler_params=pltpu.CompilerParams(
            dimension_semantics=("parallel",)),
    )(qagents, state, w_cat, expand, b2w, b2b)
    return out.reshape(-1)[:B]


# speedup vs baseline: 1.4873x; 1.4873x over previous
import functools

import jax
import jax.numpy as jnp
from jax import lax
from jax.experimental import pallas as pl
from jax.experimental.pallas import tpu as pltpu

_TR = 256


def _qmix_block(bb, tr, qt_ref, s_ref, wcat_ref, qw_ref, b2w_ref, b2b_ref,
                out_ref):
    f32 = jnp.float32
    bf16 = jnp.bfloat16
    nt = bb // tr

    wc = wcat_ref[...]
    w = jnp.concatenate([
        wc[:, 0:512],
        wc[:, 512:576],
        0.5 * wc[:, 640:704],
        wc[:, 768:896],
    ], axis=1).astype(bf16)
    qw = qw_ref[...].astype(bf16)
    b2w = b2w_ref[...]
    b2b = b2b_ref[...]

    col = jax.lax.broadcasted_iota(jnp.int32, (128, nt), 1)
    emats = [jnp.where(col == t, 1.0, 0.0).astype(bf16) for t in range(nt)]

    s1 = jnp.concatenate(
        [s_ref[...].astype(bf16), jnp.ones((bb, 1), bf16)], axis=1)
    q1t = qt_ref[...].astype(bf16)
    low = jax.lax.broadcasted_iota(jnp.int32, (tr, 128), 1) < 64

    acc = jnp.zeros((tr, nt), f32)
    for t in range(nt):
        r0, r1 = t * tr, (t + 1) * tr
        p = jnp.dot(s1[r0:r1, :], w, preferred_element_type=f32)
        qe = lax.dot_general(q1t[:, r0:r1], qw,
                             (((0,), (0,)), ((), ())),
                             preferred_element_type=f32)

        y = None
        for j in range(4):
            x = jnp.abs(p[:, 128 * j:128 * (j + 1)])
            x = x * qe[:, 128 * j:128 * (j + 1)]
            y = x if y is None else y + x
        hid = y + pltpu.roll(y, 64, axis=1)

        bw = p[:, 512:640]
        r = pltpu.roll(bw, 64, axis=1)
        b1d = jnp.where(low, bw, r)
        w2d = jnp.abs(jnp.where(low, r, bw))

        mixed = hid + b1d
        mixed = jnp.where(mixed > 0.0, mixed,
                          jnp.exp(jnp.minimum(mixed, 0.0)) - 1.0)

        h2 = jnp.maximum(p[:, 640:768], 0.0)
        z = (mixed * w2d + h2 * b2w).astype(bf16)
        acc = acc + jnp.dot(z, emats[t], preferred_element_type=f32)

    acct = jnp.transpose(acc, (1, 0)) + b2b
    for t in range(nt):
        for k in range(tr // 128):
            out_ref[t * (tr // 128) + k:t * (tr // 128) + k + 1, :] = (
                acct[t:t + 1, 128 * k:128 * (k + 1)])


def kernel(qagents, state, w_cat, expand, reduce, b2w, b2b):
    del reduce
    f32 = jnp.float32
    B, A = qagents.shape
    S = state.shape[1]
    Sk, c = w_cat.shape
    w0 = expand.shape[1]

    BB = 4096 if B % 4096 == 0 else max(8, ((B + 7) // 8) * 8)
    TR = _TR if BB % _TR == 0 else BB
    grid_b = pl.cdiv(B, BB)
    b_pad = grid_b * BB
    qt = qagents.T
    if b_pad != B:
        qt = jnp.pad(qt, ((0, 0), (0, b_pad - B)))
        state = jnp.pad(state, ((0, b_pad - B), (0, 0)))

    out = pl.pallas_call(
        functools.partial(_qmix_block, BB, TR),
        out_shape=jax.ShapeDtypeStruct((b_pad // 128, 128), f32),
        grid=(grid_b,),
        in_specs=[
            pl.BlockSpec((A, BB), lambda i: (0, i)),
            pl.BlockSpec((BB, S), lambda i: (i, 0)),
            pl.BlockSpec((Sk, c), lambda i: (0, 0)),
            pl.BlockSpec((A, w0), lambda i: (0, 0)),
            pl.BlockSpec((1, 128), lambda i: (0, 0)),
            pl.BlockSpec((1, 1), lambda i: (0, 0)),
        ],
        out_specs=pl.BlockSpec((BB // 128, 128), lambda i: (i, 0)),
        compiler_params=pltpu.CompilerParams(
            dimension_semantics=("parallel",)),
    )(qt, state, w_cat, expand, b2w, b2b)
    return out.reshape(-1)[:B]


# speedup vs baseline: 1.9498x; 1.3110x over previous
import functools

import jax
import jax.numpy as jnp
from jax import lax
from jax.experimental import pallas as pl
from jax.experimental.pallas import tpu as pltpu

_TR = 256


def _qmix_block(bb, tr, qt_ref, s_ref, wcat_ref, qw_ref, b2w_ref, b2b_ref,
                out_ref):
    f32 = jnp.float32
    bf16 = jnp.bfloat16
    nt = bb // tr

    wc = wcat_ref[...]
    w = jnp.concatenate([
        wc[:, 0:512],
        wc[:, 512:576],
        wc[:, 640:704],
        wc[:, 768:896],
    ], axis=1).astype(bf16)
    qw = qw_ref[...].astype(bf16)
    b2w = b2w_ref[...]
    b2b = b2b_ref[...]

    s1 = jnp.concatenate(
        [s_ref[...].astype(bf16), jnp.ones((bb, 1), bf16)], axis=1)
    q1t = qt_ref[...].astype(bf16)
    low = jax.lax.broadcasted_iota(jnp.int32, (tr, 128), 1) < 64

    rows = jax.lax.broadcasted_iota(jnp.int32, (256, nt), 0)
    cols = jax.lax.broadcasted_iota(jnp.int32, (256, nt), 1)
    reg = jnp.bitwise_and(rows // 64, 1)

    def half(t):
        r0, r1 = t * tr, (t + 1) * tr
        p = jnp.dot(s1[r0:r1, :], w, preferred_element_type=f32)
        qe = lax.dot_general(q1t[:, r0:r1], qw,
                             (((0,), (0,)), ((), ())),
                             preferred_element_type=f32)
        y = None
        for j in range(4):
            x = jnp.abs(p[:, 128 * j:128 * (j + 1)])
            x = x * qe[:, 128 * j:128 * (j + 1)]
            y = x if y is None else y + x
        hid = y + pltpu.roll(y, 64, axis=1)
        bw = p[:, 512:640]
        bwr = pltpu.roll(bw, 64, axis=1)
        x2 = jnp.maximum(p[:, 640:768], 0.0) * b2w
        x2f = x2 + pltpu.roll(x2, 64, axis=1)
        return hid, bw, bwr, x2f

    acc = jnp.zeros((tr, nt), f32)
    for u in range(nt // 2):
        te, to = 2 * u, 2 * u + 1
        hid_e, bw_e, bwr_e, x2f_e = half(te)
        hid_o, bw_o, bwr_o, x2f_o = half(to)
        hidp = jnp.where(low, hid_e, hid_o)
        b1p = jnp.where(low, bw_e, bwr_o)
        w2p = jnp.abs(jnp.where(low, bwr_e, bw_o))
        x2p = jnp.where(low, x2f_e, x2f_o)
        mixed = hidp + b1p
        mixed = jnp.where(mixed > 0.0, mixed,
                          jnp.exp(jnp.minimum(mixed, 0.0)) - 1.0)
        z = jnp.concatenate([mixed * w2p, x2p], axis=1).astype(bf16)
        e_u = jnp.where(cols == te + reg, 1.0, 0.0).astype(bf16)
        acc = acc + jnp.dot(z, e_u, preferred_element_type=f32)

    acct = jnp.transpose(acc, (1, 0)) + b2b
    for t in range(nt):
        for k in range(tr // 128):
            out_ref[t * (tr // 128) + k:t * (tr // 128) + k + 1, :] = (
                acct[t:t + 1, 128 * k:128 * (k + 1)])


def kernel(qagents, state, w_cat, expand, reduce, b2w, b2b):
    del reduce
    f32 = jnp.float32
    B, A = qagents.shape
    S = state.shape[1]
    Sk, c = w_cat.shape
    w0 = expand.shape[1]

    BB = 4096 if B % 4096 == 0 else max(8, ((B + 7) // 8) * 8)
    TR = _TR if BB % (2 * _TR) == 0 else BB
    grid_b = pl.cdiv(B, BB)
    b_pad = grid_b * BB
    qt = qagents.T
    if b_pad != B:
        qt = jnp.pad(qt, ((0, 0), (0, b_pad - B)))
        state = jnp.pad(state, ((0, b_pad - B), (0, 0)))

    out = pl.pallas_call(
        functools.partial(_qmix_block, BB, TR),
        out_shape=jax.ShapeDtypeStruct((b_pad // 128, 128), f32),
        grid=(grid_b,),
        in_specs=[
            pl.BlockSpec((A, BB), lambda i: (0, i)),
            pl.BlockSpec((BB, S), lambda i: (i, 0)),
            pl.BlockSpec((Sk, c), lambda i: (0, 0)),
            pl.BlockSpec((A, w0), lambda i: (0, 0)),
            pl.BlockSpec((1, 128), lambda i: (0, 0)),
            pl.BlockSpec((1, 1), lambda i: (0, 0)),
        ],
        out_specs=pl.BlockSpec((BB // 128, 128), lambda i: (i, 0)),
        compiler_params=pltpu.CompilerParams(
            dimension_semantics=("parallel",)),
    )(qt, state, w_cat, expand, b2w, b2b)
    return out.reshape(-1)[:B]


# speedup vs baseline: 1.9823x; 1.0167x over previous
import functools

import jax
import jax.numpy as jnp
from jax import lax
from jax.experimental import pallas as pl
from jax.experimental.pallas import tpu as pltpu

_TR = 256


def _qmix_block(bb, tr, qt_ref, s_ref, wcat_ref, qw_ref, b2w_ref, b2b_ref,
                out_ref):
    f32 = jnp.float32
    bf16 = jnp.bfloat16
    nt = bb // tr

    wc = wcat_ref[...]
    w = jnp.concatenate([
        wc[:, 0:512],
        wc[:, 512:576],
        wc[:, 640:704],
        wc[:, 768:896],
    ], axis=1).astype(bf16)
    qw = qw_ref[...].astype(bf16)
    b2w = b2w_ref[...]
    b2b = b2b_ref[...]

    s1 = jnp.concatenate(
        [s_ref[...].astype(bf16), jnp.ones((bb, 1), bf16)], axis=1)
    q1t = qt_ref[...].astype(bf16)
    low = jax.lax.broadcasted_iota(jnp.int32, (tr, 128), 1) < 64

    rows = jax.lax.broadcasted_iota(jnp.int32, (256, nt), 0)
    cols = jax.lax.broadcasted_iota(jnp.int32, (256, nt), 1)
    reg = jnp.bitwise_and(rows // 64, 1)

    def half(t):
        r0, r1 = t * tr, (t + 1) * tr
        p = jnp.dot(s1[r0:r1, :], w, preferred_element_type=f32)
        qe = lax.dot_general(q1t[:, r0:r1], qw,
                             (((0,), (0,)), ((), ())),
                             preferred_element_type=f32)
        y = None
        for j in range(4):
            x = jnp.abs(p[:, 128 * j:128 * (j + 1)])
            x = x * qe[:, 128 * j:128 * (j + 1)]
            y = x if y is None else y + x
        hid = y + pltpu.roll(y, 64, axis=1)
        bw = p[:, 512:640]
        bwr = pltpu.roll(bw, 64, axis=1)
        x2 = jnp.maximum(p[:, 640:768], 0.0) * b2w
        x2f = x2 + pltpu.roll(x2, 64, axis=1)
        return hid, bw, bwr, x2f

    acc = jnp.zeros((tr, nt), f32)
    for u in range(nt // 2):
        te, to = 2 * u, 2 * u + 1
        hid_e, bw_e, bwr_e, x2f_e = half(te)
        hid_o, bw_o, bwr_o, x2f_o = half(to)
        hidp = jnp.where(low, hid_e, hid_o)
        b1p = jnp.where(low, bw_e, bwr_o)
        w2p = jnp.abs(jnp.where(low, bwr_e, bw_o))
        x2p = jnp.where(low, x2f_e, x2f_o)
        mixed = hidp + b1p
        mixed = jnp.where(mixed > 0.0, mixed,
                          jnp.exp(jnp.minimum(mixed, 0.0)) - 1.0)
        z = jnp.concatenate([mixed * w2p, x2p], axis=1).astype(bf16)
        e_u = jnp.where(cols == te + reg, 1.0, 0.0).astype(bf16)
        acc = acc + jnp.dot(z, e_u, preferred_element_type=f32)

    acct = jnp.transpose(acc, (1, 0)) + b2b
    for t in range(nt):
        for k in range(tr // 128):
            out_ref[t * (tr // 128) + k:t * (tr // 128) + k + 1, :] = (
                acct[t:t + 1, 128 * k:128 * (k + 1)])


def kernel(qagents, state, w_cat, expand, reduce, b2w, b2b):
    del reduce
    f32 = jnp.float32
    B, A = qagents.shape
    S = state.shape[1]
    Sk, c = w_cat.shape
    w0 = expand.shape[1]

    BB = 8192 if B % 8192 == 0 else max(8, ((B + 7) // 8) * 8)
    TR = _TR if BB % (2 * _TR) == 0 else BB
    grid_b = pl.cdiv(B, BB)
    b_pad = grid_b * BB
    qt = qagents.T
    if b_pad != B:
        qt = jnp.pad(qt, ((0, 0), (0, b_pad - B)))
        state = jnp.pad(state, ((0, b_pad - B), (0, 0)))

    out = pl.pallas_call(
        functools.partial(_qmix_block, BB, TR),
        out_shape=jax.ShapeDtypeStruct((b_pad // 128, 128), f32),
        grid=(grid_b,),
        in_specs=[
            pl.BlockSpec((A, BB), lambda i: (0, i)),
            pl.BlockSpec((BB, S), lambda i: (i, 0)),
            pl.BlockSpec((Sk, c), lambda i: (0, 0)),
            pl.BlockSpec((A, w0), lambda i: (0, 0)),
            pl.BlockSpec((1, 128), lambda i: (0, 0)),
            pl.BlockSpec((1, 1), lambda i: (0, 0)),
        ],
        out_specs=pl.BlockSpec((BB // 128, 128), lambda i: (i, 0)),
        compiler_params=pltpu.CompilerParams(
            dimension_semantics=("parallel",)),
    )(qt, state, w_cat, expand, b2w, b2b)
    return out.reshape(-1)[:B]


# speedup vs baseline: 4.5895x; 2.3152x over previous
import functools

import jax
import jax.numpy as jnp
from jax import lax
from jax.experimental import pallas as pl
from jax.experimental.pallas import tpu as pltpu

_TR = 256


def _qmix_block(bb, tr, qt_ref, s_ref, wcat_ref, b2w_ref, b2b_ref, out_ref):
    f32 = jnp.float32
    bf16 = jnp.bfloat16
    nt = bb // tr
    nk = tr // 128

    wc = wcat_ref[...]
    w = jnp.concatenate([
        wc[:, 0:512],
        wc[:, 512:576],
        wc[:, 640:704],
        wc[:, 768:896],
    ], axis=1).astype(bf16)
    b2wt = jnp.transpose(b2w_ref[...], (1, 0))
    b2b = b2b_ref[...]

    s1 = jnp.concatenate(
        [s_ref[...].astype(bf16), jnp.ones((bb, 1), bf16)], axis=1)
    qt = qt_ref[...]
    lowr = jax.lax.broadcasted_iota(jnp.int32, (128, tr), 0) < 64

    def half(t):
        c0, c1 = t * tr, (t + 1) * tr
        pt = lax.dot_general(w, s1[c0:c1, :],
                             (((0,), (1,)), ((), ())),
                             preferred_element_type=f32)
        y = None
        for j in range(4):
            x = jnp.abs(pt[128 * j:128 * (j + 1), :])
            qs = jnp.where(lowr, qt[2 * j:2 * j + 1, c0:c1],
                           qt[2 * j + 1:2 * j + 2, c0:c1])
            x = x * qs
            y = x if y is None else y + x
        hid = y + pltpu.roll(y, 64, axis=0)
        bw = pt[512:640, :]
        bwr = pltpu.roll(bw, 64, axis=0)
        x2 = jnp.maximum(pt[640:768, :], 0.0) * b2wt
        x2f = x2 + pltpu.roll(x2, 64, axis=0)
        return hid, bw, bwr, x2f

    for u in range(nt // 2):
        te, to = 2 * u, 2 * u + 1
        hid_e, bw_e, bwr_e, x2f_e = half(te)
        hid_o, bw_o, bwr_o, x2f_o = half(to)
        hidp = jnp.where(lowr, hid_e, hid_o)
        b1p = jnp.where(lowr, bw_e, bwr_o)
        w2p = jnp.abs(jnp.where(lowr, bwr_e, bw_o))
        x2p = jnp.where(lowr, x2f_e, x2f_o)
        mixed = hidp + b1p
        mixed = jnp.where(mixed > 0.0, mixed,
                          jnp.exp(jnp.minimum(mixed, 0.0)) - 1.0)
        full = mixed * w2p + x2p
        qe = jnp.sum(full[0:64, :], axis=0, keepdims=True) + b2b
        qo = jnp.sum(full[64:128, :], axis=0, keepdims=True) + b2b
        for k in range(nk):
            out_ref[te * nk + k:te * nk + k + 1, :] = (
                qe[:, 128 * k:128 * (k + 1)])
            out_ref[to * nk + k:to * nk + k + 1, :] = (
                qo[:, 128 * k:128 * (k + 1)])


def kernel(qagents, state, w_cat, expand, reduce, b2w, b2b):
    del expand, reduce
    f32 = jnp.float32
    B, A = qagents.shape
    S = state.shape[1]
    Sk, c = w_cat.shape

    BB = 8192 if B % 8192 == 0 else max(8, ((B + 7) // 8) * 8)
    TR = _TR if BB % (2 * _TR) == 0 else BB
    grid_b = pl.cdiv(B, BB)
    b_pad = grid_b * BB
    qt = qagents.T
    if b_pad != B:
        qt = jnp.pad(qt, ((0, 0), (0, b_pad - B)))
        state = jnp.pad(state, ((0, b_pad - B), (0, 0)))

    out = pl.pallas_call(
        functools.partial(_qmix_block, BB, TR),
        out_shape=jax.ShapeDtypeStruct((b_pad // 128, 128), f32),
        grid=(grid_b,),
        in_specs=[
            pl.BlockSpec((A, BB), lambda i: (0, i)),
            pl.BlockSpec((BB, S), lambda i: (i, 0)),
            pl.BlockSpec((Sk, c), lambda i: (0, 0)),
            pl.BlockSpec((1, 128), lambda i: (0, 0)),
            pl.BlockSpec((1, 1), lambda i: (0, 0)),
        ],
        out_specs=pl.BlockSpec((BB // 128, 128), lambda i: (i, 0)),
        compiler_params=pltpu.CompilerParams(
            dimension_semantics=("parallel",)),
    )(qt, state, w_cat, b2w, b2b)
    return out.reshape(-1)[:B]


# speedup vs baseline: 4.6063x; 1.0037x over previous
import functools

import jax
import jax.numpy as jnp
from jax import lax
from jax.experimental import pallas as pl
from jax.experimental.pallas import tpu as pltpu

_TR = 256


def _qmix_block(bb, tr, qt_ref, s_ref, wcat_ref, b2w_ref, b2b_ref, out_ref):
    f32 = jnp.float32
    bf16 = jnp.bfloat16
    nt = bb // tr
    nk = tr // 128

    wc = wcat_ref[...]
    w = jnp.concatenate([
        wc[:, 0:512],
        wc[:, 512:576],
        wc[:, 640:704],
        wc[:, 768:896],
    ], axis=1).astype(bf16)
    b2wt = jnp.transpose(b2w_ref[...], (1, 0))
    b2b = b2b_ref[...]

    s1 = jnp.concatenate(
        [s_ref[...].astype(bf16), jnp.ones((bb, 1), bf16)], axis=1)
    qt = qt_ref[...]
    lowr = jax.lax.broadcasted_iota(jnp.int32, (128, tr), 0) < 64

    def half(t):
        c0, c1 = t * tr, (t + 1) * tr
        pt = lax.dot_general(w, s1[c0:c1, :],
                             (((0,), (1,)), ((), ())),
                             preferred_element_type=f32)
        y = None
        for j in range(4):
            x = jnp.abs(pt[128 * j:128 * (j + 1), :])
            qs = jnp.where(lowr, qt[2 * j:2 * j + 1, c0:c1],
                           qt[2 * j + 1:2 * j + 2, c0:c1])
            x = x * qs
            y = x if y is None else y + x
        hid = y + pltpu.roll(y, 64, axis=0)
        bw = pt[512:640, :]
        bwr = pltpu.roll(bw, 64, axis=0)
        x2 = jnp.maximum(pt[640:768, :], 0.0) * b2wt
        x2f = x2 + pltpu.roll(x2, 64, axis=0)
        return hid, bw, bwr, x2f

    for u in range(nt // 2):
        te, to = 2 * u, 2 * u + 1
        hid_e, bw_e, bwr_e, x2f_e = half(te)
        hid_o, bw_o, bwr_o, x2f_o = half(to)
        hidp = jnp.where(lowr, hid_e, hid_o)
        b1p = jnp.where(lowr, bw_e, bwr_o)
        w2p = jnp.abs(jnp.where(lowr, bwr_e, bw_o))
        x2p = jnp.where(lowr, x2f_e, x2f_o)
        mixed = hidp + b1p
        mixed = jnp.where(mixed > 0.0, mixed,
                          jnp.exp(jnp.minimum(mixed, 0.0)) - 1.0)
        full = mixed * w2p + x2p
        qe = jnp.sum(full[0:64, :], axis=0, keepdims=True) + b2b
        qo = jnp.sum(full[64:128, :], axis=0, keepdims=True) + b2b
        for k in range(nk):
            out_ref[te * nk + k:te * nk + k + 1, :] = (
                qe[:, 128 * k:128 * (k + 1)])
            out_ref[to * nk + k:to * nk + k + 1, :] = (
                qo[:, 128 * k:128 * (k + 1)])


def kernel(qagents, state, w_cat, expand, reduce, b2w, b2b):
    del expand, reduce
    f32 = jnp.float32
    B, A = qagents.shape
    S = state.shape[1]
    Sk, c = w_cat.shape

    BB = 16384 if B % 16384 == 0 else max(8, ((B + 7) // 8) * 8)
    TR = _TR if BB % (2 * _TR) == 0 else BB
    grid_b = pl.cdiv(B, BB)
    b_pad = grid_b * BB
    qt = qagents.T
    if b_pad != B:
        qt = jnp.pad(qt, ((0, 0), (0, b_pad - B)))
        state = jnp.pad(state, ((0, b_pad - B), (0, 0)))

    out = pl.pallas_call(
        functools.partial(_qmix_block, BB, TR),
        out_shape=jax.ShapeDtypeStruct((b_pad // 128, 128), f32),
        grid=(grid_b,),
        in_specs=[
            pl.BlockSpec((A, BB), lambda i: (0, i)),
            pl.BlockSpec((BB, S), lambda i: (i, 0)),
            pl.BlockSpec((Sk, c), lambda i: (0, 0)),
            pl.BlockSpec((1, 128), lambda i: (0, 0)),
            pl.BlockSpec((1, 1), lambda i: (0, 0)),
        ],
        out_specs=pl.BlockSpec((BB // 128, 128), lambda i: (i, 0)),
        compiler_params=pltpu.CompilerParams(
            dimension_semantics=("parallel",)),
    )(qt, state, w_cat, b2w, b2b)
    return out.reshape(-1)[:B]
